# baseline (speedup 1.0000x reference)
import functools

import jax
import jax.numpy as jnp
from jax import lax
from jax.experimental import pallas as pl
from jax.experimental.pallas import tpu as pltpu
from jax.experimental.pallas import tpu_sc as plsc

_E = 16
_ROWS = 8192
_COLS = 2048
_INV_N = 1.0 / float(_ROWS * _COLS)
_ALPHA = 1.0 / 1000.0
_Z = 0.001

_SC_ROWS = 1536
_TC_ROWS = _ROWS - _SC_ROWS
_NTILES = 32
_PER_TILE = _SC_ROWS * _COLS // _NTILES
_SC_START = _TC_ROWS * _COLS
_CH = 16384
_NCHUNK = _PER_TILE // _CH

_BLK = 256
_NB = _TC_ROWS // _BLK
_GRID = _NB // 2


def _tc_sum_body(a_ref, b_ref, out_ref, acc_ref):
    i = pl.program_id(0)

    @pl.when(i == 0)
    def _init():
        acc_ref[0] = 0.0

    acc_ref[0] += jnp.sum(a_ref[...]) + jnp.sum(b_ref[...])

    @pl.when(i == _GRID - 1)
    def _fin():
        ids = lax.broadcasted_iota(jnp.int32, (1, _E), 1)
        out_ref[...] = jnp.where(ids == 0, acc_ref[0], 0.0)


def _tc_partial_sum(wm_state):
    wm3d = wm_state.reshape(_ROWS // _BLK, _BLK, _COLS)
    return pl.pallas_call(
        _tc_sum_body,
        grid=(_GRID,),
        in_specs=[
            pl.BlockSpec((_BLK, _COLS), lambda i: (2 * i, 0)),
            pl.BlockSpec((1, _BLK, _COLS), lambda i: (2 * i + 1, 0, 0)),
        ],
        out_specs=pl.BlockSpec((1, _E), lambda i: (0, 0)),
        out_shape=jax.ShapeDtypeStruct((1, _E), jnp.float32),
        scratch_shapes=[pltpu.SMEM((1,), jnp.float32)],
    )(wm_state, wm3d)


_SC_MESH = plsc.VectorSubcoreMesh(core_axis_name="c", subcore_axis_name="s")


@functools.partial(
    pl.kernel,
    out_type=jax.ShapeDtypeStruct((_NTILES, _E), jnp.float32),
    mesh=_SC_MESH,
    compiler_params=pltpu.CompilerParams(needs_layout_passes=False),
    scratch_types=[
        pltpu.VMEM((_CH,), jnp.float32),
        pltpu.VMEM((_CH,), jnp.float32),
        pltpu.VMEM((_E,), jnp.float32),
        pltpu.SemaphoreType.DMA,
        pltpu.SemaphoreType.DMA,
    ],
)
def _sc_partial_sums(wm_hbm, parts_hbm, b0, b1, v_out, s0, s1):
    cid = lax.axis_index("c")
    sid = lax.axis_index("s")
    wid = sid * 2 + cid
    base = _SC_START + wid * _PER_TILE

    bufs = (b0, b1)
    sems = (s0, s1)
    _U = 8
    _NBODY = _CH // (_E * _U)

    accs = (jnp.zeros((_E,), jnp.float32),) * 4
    pending = pltpu.async_copy(wm_hbm.at[pl.ds(base, _CH)], b0, s0)
    for k in range(_NCHUNK):
        nxt = None
        if k + 1 < _NCHUNK:
            nxt = pltpu.async_copy(
                wm_hbm.at[pl.ds(base + (k + 1) * _CH, _CH)],
                bufs[(k + 1) % 2], sems[(k + 1) % 2])
        pending.wait()
        buf = bufs[k % 2]

        def _body(j, accs, buf=buf):
            a0, a1, a2, a3 = accs
            off = j * (_E * _U)
            v = [buf[pl.ds(off + u * _E, _E)] for u in range(_U)]
            return (a0 + v[0] + v[4], a1 + v[1] + v[5],
                    a2 + v[2] + v[6], a3 + v[3] + v[7])

        accs = lax.fori_loop(0, _NBODY, _body, accs)
        pending = nxt

    v_out[...] = (accs[0] + accs[1]) + (accs[2] + accs[3])
    pltpu.sync_copy(v_out, parts_hbm.at[wid])


def _combine_body(tc_ref, parts_ref, proto_ref, ema_ref,
                  mask_ref, probs_ref, loss_ref, idx_ref, usage_ref):
    total = jnp.sum(parts_ref[...]) + jnp.sum(tc_ref[...])
    x = total * _INV_N
    ids = lax.broadcasted_iota(jnp.int32, (1, _E), 1)
    p = proto_ref[...]
    sim = -((p - x) ** 2)
    m = jnp.max(sim)
    e = jnp.exp(sim - m)
    probs = e / jnp.sum(e)
    m1 = jnp.max(probs)
    i1 = jnp.min(jnp.where(probs == m1, ids, _E))
    hit1 = ids == i1
    probs2 = jnp.where(hit1, -jnp.inf, probs)
    m2 = jnp.max(probs2)
    i2 = jnp.min(jnp.where(probs2 == m2, ids, _E))
    mask = (hit1 | (ids == i2)).astype(jnp.float32)
    usage = (1.0 - _ALPHA) * ema_ref[...] + _ALPHA * mask
    d = usage - (1.0 / _E)
    loss = jnp.sum(d * d) * (1.0 / _E) * _Z
    mask_ref[...] = mask
    probs_ref[...] = probs
    loss_ref[...] = jnp.full((1, _E), loss, jnp.float32)
    idx_ref[...] = jnp.where(ids == 0, i1, jnp.where(ids == 1, i2, 0))
    usage_ref[...] = usage


def _combine(tc16, sc_parts, proto2d, ema2d):
    full = pl.BlockSpec((1, _E), lambda: (0, 0))
    return pl.pallas_call(
        _combine_body,
        in_specs=[full, pl.BlockSpec((_NTILES, _E), lambda: (0, 0)),
                  full, full],
        out_specs=[full, full, full, full, full],
        out_shape=[
            jax.ShapeDtypeStruct((1, _E), jnp.float32),
            jax.ShapeDtypeStruct((1, _E), jnp.float32),
            jax.ShapeDtypeStruct((1, _E), jnp.float32),
            jax.ShapeDtypeStruct((1, _E), jnp.int32),
            jax.ShapeDtypeStruct((1, _E), jnp.float32),
        ],
    )(tc16, sc_parts, proto2d, ema2d)


@jax.jit
def kernel(wm_state, prototypes, usage_ema):
    wm_flat = wm_state.reshape(_ROWS * _COLS)
    sc_parts = _sc_partial_sums(wm_flat)
    tc16 = _tc_partial_sum(wm_state)
    mask2d, probs2d, loss2d, idx2d, usage2d = _combine(
        tc16, sc_parts, prototypes.reshape(1, _E), usage_ema.reshape(1, _E))
    return (mask2d[0], probs2d[0], loss2d[0, 0], idx2d[0, :2], usage2d[0])


# speedup vs baseline: 3.6558x; 3.6558x over previous
import jax
import jax.numpy as jnp
from jax import lax
from jax.experimental import pallas as pl
from jax.experimental.pallas import tpu as pltpu

_E = 16
_ROWS = 8192
_COLS = 2048
_BLK = 512
_GRID = _ROWS // (2 * _BLK)
_INV_N = 1.0 / float(_ROWS * _COLS)
_ALPHA = 1.0 / 1000.0
_Z = 0.001


def _router_kernel(a_ref, b_ref, proto_ref, ema_ref,
                   mask_ref, probs_ref, loss_ref, idx_ref, usage_ref,
                   acc_ref):
    i = pl.program_id(0)

    @pl.when(i == 0)
    def _init():
        acc_ref[0] = 0.0

    acc_ref[0] += jnp.sum(a_ref[...]) + jnp.sum(b_ref[...])

    @pl.when(i == _GRID - 1)
    def _epilogue():
        x = acc_ref[0] * _INV_N
        ids = lax.broadcasted_iota(jnp.int32, (1, _E), 1)
        p = proto_ref[...]
        sim = -((p - x) ** 2)
        m = jnp.max(sim)
        e = jnp.exp(sim - m)
        probs = e / jnp.sum(e)
        m1 = jnp.max(probs)
        i1 = jnp.min(jnp.where(probs == m1, ids, _E))
        hit1 = ids == i1
        probs2 = jnp.where(hit1, -jnp.inf, probs)
        m2 = jnp.max(probs2)
        i2 = jnp.min(jnp.where(probs2 == m2, ids, _E))
        mask = (hit1 | (ids == i2)).astype(jnp.float32)
        usage = (1.0 - _ALPHA) * ema_ref[...] + _ALPHA * mask
        d = usage - (1.0 / _E)
        loss = jnp.sum(d * d) * (1.0 / _E) * _Z
        mask_ref[...] = mask
        probs_ref[...] = probs
        loss_ref[...] = jnp.full((1, _E), loss, jnp.float32)
        idx_ref[...] = jnp.where(ids == 0, i1, jnp.where(ids == 1, i2, 0))
        usage_ref[...] = usage


@jax.jit
def kernel(wm_state, prototypes, usage_ema):
    wm3d = wm_state.reshape(_ROWS // _BLK, _BLK, _COLS)
    full = pl.BlockSpec((1, _E), lambda i: (0, 0))
    outs = pl.pallas_call(
        _router_kernel,
        grid=(_GRID,),
        in_specs=[
            pl.BlockSpec((_BLK, _COLS), lambda i: (2 * i, 0)),
            pl.BlockSpec((1, _BLK, _COLS), lambda i: (2 * i + 1, 0, 0)),
            full,
            full,
        ],
        out_specs=[full, full, full, full, full],
        out_shape=[
            jax.ShapeDtypeStruct((1, _E), jnp.float32),
            jax.ShapeDtypeStruct((1, _E), jnp.float32),
            jax.ShapeDtypeStruct((1, _E), jnp.float32),
            jax.ShapeDtypeStruct((1, _E), jnp.int32),
            jax.ShapeDtypeStruct((1, _E), jnp.float32),
        ],
        scratch_shapes=[pltpu.SMEM((1,), jnp.float32)],
    )(wm_state, wm3d, prototypes.reshape(1, _E), usage_ema.reshape(1, _E))
    mask2d, probs2d, loss2d, idx2d, usage2d = outs
    return (mask2d[0], probs2d[0], loss2d[0, 0], idx2d[0, :2], usage2d[0])
